# Initial kernel scaffold; baseline (speedup 1.0000x reference)
import jax
import jax.numpy as jnp
from jax.experimental import pallas as pl


def kernel(x, edge_index, W1, b1, W2, b2, fc1_W, fc1_b, fc2_W, fc2_b):
    raise NotImplementedError("write your pallas kernel here")


# speedup vs baseline: 8.2225x; 8.2225x over previous
import dataclasses
import functools

import jax
import jax.numpy as jnp
from jax import lax
from jax.experimental import pallas as pl
from jax.experimental.pallas import tpu as pltpu
from jax.experimental.pallas import tpu_sc as plsc

_NC = 2
_NS = 16
_LN = 16
_CH = 128
_IDXB = 8
_BLK = 2048


def _sc_compiler_params():
    cp = pltpu.CompilerParams()
    if "needs_layout_passes" in pltpu.CompilerParams.__dataclass_fields__:
        cp = dataclasses.replace(cp, needs_layout_passes=False)
    return cp


def _mm(a, b):
    return lax.dot_general(a, b, (((1,), (0,)), ((), ())),
                           precision=lax.Precision.HIGHEST,
                           preferred_element_type=jnp.float32)


def _sc_degree(col2d, n_pad):
    chunks = col2d.shape[0]
    per_tile = chunks // (_NC * _NS)
    mesh = plsc.VectorSubcoreMesh(core_axis_name="c", subcore_axis_name="s")

    @functools.partial(
        pl.kernel,
        out_type=jax.ShapeDtypeStruct((_NC * _NS, n_pad), jnp.float32),
        mesh=mesh,
        compiler_params=_sc_compiler_params(),
        scratch_types=[
            pltpu.VMEM((n_pad,), jnp.float32),
            pltpu.VMEM((_IDXB, _CH), jnp.int32),
        ],
    )
    def k(col_hbm, out_hbm, hist, idx):
        c = lax.axis_index("c")
        s = lax.axis_index("s")
        wid = s * _NC + c
        zeros = jnp.zeros((_LN,), jnp.float32)
        ones = jnp.ones((_LN,), jnp.float32)

        @pl.loop(0, n_pad // _LN)
        def _(i):
            hist[pl.ds(i * _LN, _LN)] = zeros

        base = wid * per_tile

        @pl.loop(0, per_tile // _IDXB)
        def _(b):
            pltpu.sync_copy(col_hbm.at[pl.ds(base + b * _IDXB, _IDXB)], idx)

            @pl.loop(0, _IDXB)
            def _(j):
                @pl.loop(0, _CH // _LN)
                def _(q):
                    idx16 = idx[j, pl.ds(q * _LN, _LN)]
                    plsc.addupdate_scatter(hist, [idx16], ones)

        pltpu.sync_copy(hist, out_hbm.at[wid])

    return k(col2d)


def _sc_propagate(row2d, col2d, g, n_pad):
    chunks = row2d.shape[0]
    per_tile = chunks // (_NC * _NS)
    d = g.shape[1]
    zrows = n_pad // _NS
    mesh = plsc.VectorSubcoreMesh(core_axis_name="c", subcore_axis_name="s")

    @functools.partial(
        pl.kernel,
        out_type=jax.ShapeDtypeStruct((_NC, n_pad, d), jnp.float32),
        mesh=mesh,
        scratch_types=[
            pltpu.VMEM((_IDXB, _CH), jnp.int32),
            pltpu.VMEM((_IDXB, _CH), jnp.int32),
            pltpu.VMEM((_CH, d), jnp.float32),
            pltpu.VMEM((_CH, d), jnp.float32),
            pltpu.VMEM_SHARED((n_pad, d), jnp.float32),
            pltpu.SemaphoreType.DMA,
        ],
    )
    def k(row_hbm, col_hbm, g_hbm, out_hbm, ridx, cidx, gbuf, zbuf, acc, sem):
        c = lax.axis_index("c")
        s = lax.axis_index("s")
        zeros = jnp.zeros((_LN,), jnp.float32)

        @pl.loop(0, _CH)
        def _(i):
            @pl.loop(0, d // _LN)
            def _(j):
                zbuf[i, pl.ds(j * _LN, _LN)] = zeros

        @pl.loop(0, zrows // _CH)
        def _(i):
            pltpu.sync_copy(zbuf, acc.at[pl.ds(s * zrows + i * _CH, _CH)])

        plsc.subcore_barrier()

        base = (s * _NC + c) * per_tile

        @pl.loop(0, per_tile // _IDXB)
        def _(b):
            pltpu.sync_copy(row_hbm.at[pl.ds(base + b * _IDXB, _IDXB)], ridx)
            pltpu.sync_copy(col_hbm.at[pl.ds(base + b * _IDXB, _IDXB)], cidx)

            @pl.loop(0, _IDXB)
            def _(j):
                pltpu.async_copy(g_hbm.at[ridx.at[j]], gbuf, sem).wait()
                pltpu.sync_copy(gbuf, acc.at[cidx.at[j]], add=True)

        plsc.subcore_barrier()
        pltpu.sync_copy(acc.at[pl.ds(s * zrows, zrows)],
                        out_hbm.at[c].at[pl.ds(s * zrows, zrows)])

    return k(row2d, col2d, g)


def _tc_prep(x, w1, parts):
    n, d = x.shape
    grid = n // _BLK

    def body(x_ref, w_ref, p_ref, g_ref, dis_ref):
        deg = jnp.sum(p_ref[...], axis=0) + 1.0
        dis = lax.rsqrt(deg)
        g_ref[...] = _mm(x_ref[...], w_ref[...]) * dis[:, None]
        dis_ref[...] = dis[:, None]

    return pl.pallas_call(
        body,
        grid=(grid,),
        in_specs=[
            pl.BlockSpec((_BLK, d), lambda i: (i, 0)),
            pl.BlockSpec((d, d), lambda i: (0, 0)),
            pl.BlockSpec((_NC * _NS, _BLK), lambda i: (0, i)),
        ],
        out_specs=[
            pl.BlockSpec((_BLK, d), lambda i: (i, 0)),
            pl.BlockSpec((_BLK, 1), lambda i: (i, 0)),
        ],
        out_shape=[
            jax.ShapeDtypeStruct((n, d), jnp.float32),
            jax.ShapeDtypeStruct((n, 1), jnp.float32),
        ],
    )(x, w1, parts)


def _tc_mid(acc, g1, dis, b1, w2):
    _, n, d = acc.shape
    grid = n // _BLK

    def body(a_ref, g_ref, dis_ref, b_ref, w_ref, o_ref):
        comb = a_ref[0] + a_ref[1] + g_ref[...]
        dis = dis_ref[...]
        z = jnp.maximum(comb * dis + b_ref[...], 0.0)
        o_ref[...] = _mm(z, w_ref[...]) * dis

    return pl.pallas_call(
        body,
        grid=(grid,),
        in_specs=[
            pl.BlockSpec((2, _BLK, d), lambda i: (0, i, 0)),
            pl.BlockSpec((_BLK, d), lambda i: (i, 0)),
            pl.BlockSpec((_BLK, 1), lambda i: (i, 0)),
            pl.BlockSpec((1, d), lambda i: (0, 0)),
            pl.BlockSpec((d, d), lambda i: (0, 0)),
        ],
        out_specs=pl.BlockSpec((_BLK, d), lambda i: (i, 0)),
        out_shape=jax.ShapeDtypeStruct((n, d), jnp.float32),
    )(acc, g1, dis, b1, w2)


def _tc_final(acc, g2, dis, b2, fc1w, fc1b, fc2w, fc2b):
    _, n, d = acc.shape
    dout = fc2w.shape[1]
    grid = n // _BLK

    def body(a_ref, g_ref, dis_ref, b_ref, w1_ref, b1_ref, w2_ref, b2_ref,
             zs_ref, res_ref):
        comb = a_ref[0] + a_ref[1] + g_ref[...]
        zs = comb * dis_ref[...] + b_ref[...]
        zs_ref[...] = zs
        hh = _mm(zs, w1_ref[...]) + b1_ref[...]
        h = jnp.where(hh > 0, hh, jnp.exp(jnp.minimum(hh, 0.0)) - 1.0)
        t = _mm(h, w2_ref[...]) + b2_ref[...]
        m = jnp.max(t, axis=1, keepdims=True)
        lse = m + jnp.log(jnp.sum(jnp.exp(t - m), axis=1, keepdims=True))
        res_ref[...] = t - lse

    return pl.pallas_call(
        body,
        grid=(grid,),
        in_specs=[
            pl.BlockSpec((2, _BLK, d), lambda i: (0, i, 0)),
            pl.BlockSpec((_BLK, d), lambda i: (i, 0)),
            pl.BlockSpec((_BLK, 1), lambda i: (i, 0)),
            pl.BlockSpec((1, d), lambda i: (0, 0)),
            pl.BlockSpec((d, d), lambda i: (0, 0)),
            pl.BlockSpec((1, d), lambda i: (0, 0)),
            pl.BlockSpec((d, dout), lambda i: (0, 0)),
            pl.BlockSpec((1, dout), lambda i: (0, 0)),
        ],
        out_specs=[
            pl.BlockSpec((_BLK, d), lambda i: (i, 0)),
            pl.BlockSpec((_BLK, dout), lambda i: (i, 0)),
        ],
        out_shape=[
            jax.ShapeDtypeStruct((n, d), jnp.float32),
            jax.ShapeDtypeStruct((n, dout), jnp.float32),
        ],
    )(acc, g2, dis, b2, fc1w, fc1b, fc2w, fc2b)


def kernel(x, edge_index, W1, b1, W2, b2, fc1_W, fc1_b, fc2_W, fc2_b):
    n, d = x.shape
    e = edge_index.shape[1]
    n_pad = -(-n // (_NS * _CH)) * (_NS * _CH)

    align = _NC * _NS * _CH * _IDXB
    ep = -(-e // align) * align
    pad = ep - e
    row = jnp.concatenate([edge_index[0], jnp.zeros((pad,), edge_index.dtype)])
    col = jnp.concatenate([edge_index[1], jnp.full((pad,), n, edge_index.dtype)])
    row2d = row.reshape(-1, _CH)
    col2d = col.reshape(-1, _CH)
    x_p = jnp.concatenate([x, jnp.zeros((n_pad - n, d), x.dtype)], axis=0)

    parts = _sc_degree(col2d, n_pad)
    g1, dis = _tc_prep(x_p, W1, parts)
    acc1 = _sc_propagate(row2d, col2d, g1, n_pad)
    g2 = _tc_mid(acc1, g1, dis, b1.reshape(1, -1), W2)
    acc2 = _sc_propagate(row2d, col2d, g2, n_pad)
    zs, res = _tc_final(acc2, g2, dis, b2.reshape(1, -1), fc1_W,
                        fc1_b.reshape(1, -1), fc2_W, fc2_b.reshape(1, -1))
    return (zs[:n], res[:n])


# speedup vs baseline: 8.4150x; 1.0234x over previous
import dataclasses
import functools

import jax
import jax.numpy as jnp
from jax import lax
from jax.experimental import pallas as pl
from jax.experimental.pallas import tpu as pltpu
from jax.experimental.pallas import tpu_sc as plsc

_NC = 2
_NS = 16
_LN = 16
_CH = 128
_IDXB = 8
_BLK = 2048
_NBUF = 2


def _sc_compiler_params():
    cp = pltpu.CompilerParams()
    if "needs_layout_passes" in pltpu.CompilerParams.__dataclass_fields__:
        cp = dataclasses.replace(cp, needs_layout_passes=False)
    return cp


def _mm(a, b):
    return lax.dot_general(a, b, (((1,), (0,)), ((), ())),
                           precision=lax.Precision.HIGHEST,
                           preferred_element_type=jnp.float32)


def _sc_degree(col2d, n_pad):
    chunks = col2d.shape[0]
    per_tile = chunks // (_NC * _NS)
    mesh = plsc.VectorSubcoreMesh(core_axis_name="c", subcore_axis_name="s")

    @functools.partial(
        pl.kernel,
        out_type=jax.ShapeDtypeStruct((_NC * _NS, n_pad), jnp.float32),
        mesh=mesh,
        compiler_params=_sc_compiler_params(),
        scratch_types=[
            pltpu.VMEM((n_pad,), jnp.float32),
            pltpu.VMEM((_IDXB, _CH), jnp.int32),
        ],
    )
    def k(col_hbm, out_hbm, hist, idx):
        c = lax.axis_index("c")
        s = lax.axis_index("s")
        wid = s * _NC + c
        zeros = jnp.zeros((_LN,), jnp.float32)
        ones = jnp.ones((_LN,), jnp.float32)

        @pl.loop(0, n_pad // _LN)
        def _(i):
            hist[pl.ds(i * _LN, _LN)] = zeros

        base = wid * per_tile

        @pl.loop(0, per_tile // _IDXB)
        def _(b):
            pltpu.sync_copy(col_hbm.at[pl.ds(base + b * _IDXB, _IDXB)], idx)

            @pl.loop(0, _IDXB)
            def _(j):
                @pl.loop(0, _CH // _LN)
                def _(q):
                    idx16 = idx[j, pl.ds(q * _LN, _LN)]
                    plsc.addupdate_scatter(hist, [idx16], ones)

        pltpu.sync_copy(hist, out_hbm.at[wid])

    return k(col2d)


def _sc_propagate(row2d, col2d, g, n_pad):
    chunks = row2d.shape[0]
    per_tile = chunks // (_NC * _NS)
    d = g.shape[1]
    zrows = n_pad // _NS
    mesh = plsc.VectorSubcoreMesh(core_axis_name="c", subcore_axis_name="s")

    @functools.partial(
        pl.kernel,
        out_type=jax.ShapeDtypeStruct((_NC, n_pad, d), jnp.float32),
        mesh=mesh,
        scratch_types=[
            pltpu.VMEM((_IDXB, _CH), jnp.int32),
            pltpu.VMEM((_IDXB, _CH), jnp.int32),
            pltpu.VMEM((_NBUF, _CH, d), jnp.float32),
            pltpu.VMEM_SHARED((n_pad, d), jnp.float32),
            pltpu.SemaphoreType.DMA,
            pltpu.SemaphoreType.DMA,
        ],
    )
    def k(row_hbm, col_hbm, g_hbm, out_hbm, ridx, cidx, gbufs, acc, gsem, ssem):
        c = lax.axis_index("c")
        s = lax.axis_index("s")
        zeros = jnp.zeros((_LN,), jnp.float32)

        base = (s * _NC + c) * per_tile

        zbuf = gbufs.at[0]

        @pl.loop(0, _CH)
        def _(i):
            @pl.loop(0, d // _LN)
            def _(j):
                zbuf[i, pl.ds(j * _LN, _LN)] = zeros

        @pl.loop(0, zrows // _CH)
        def _(i):
            pltpu.sync_copy(zbuf, acc.at[pl.ds(s * zrows + i * _CH, _CH)])

        plsc.subcore_barrier()

        @pl.loop(0, per_tile // _IDXB)
        def _(bb):
            pltpu.sync_copy(row_hbm.at[pl.ds(base + bb * _IDXB, _IDXB)], ridx)
            pltpu.sync_copy(col_hbm.at[pl.ds(base + bb * _IDXB, _IDXB)], cidx)

            for j0 in range(0, _IDXB, _NBUF):
                gathers = [
                    pltpu.async_copy(g_hbm.at[ridx.at[j0 + b]], gbufs.at[b],
                                     gsem)
                    for b in range(_NBUF)
                ]
                for cp in gathers:
                    cp.wait()
                scatters = [
                    pltpu.async_copy(gbufs.at[b], acc.at[cidx.at[j0 + b]],
                                     ssem, add=True)
                    for b in range(_NBUF)
                ]
                for cp in scatters:
                    cp.wait()

        plsc.subcore_barrier()
        pltpu.sync_copy(acc.at[pl.ds(s * zrows, zrows)],
                        out_hbm.at[c].at[pl.ds(s * zrows, zrows)])

    return k(row2d, col2d, g)


def _tc_prep(x, w1, parts):
    n, d = x.shape
    grid = n // _BLK

    def body(x_ref, w_ref, p_ref, g_ref, dis_ref):
        deg = jnp.sum(p_ref[...], axis=0) + 1.0
        dis = lax.rsqrt(deg)
        g_ref[...] = _mm(x_ref[...], w_ref[...]) * dis[:, None]
        dis_ref[...] = dis[:, None]

    return pl.pallas_call(
        body,
        grid=(grid,),
        in_specs=[
            pl.BlockSpec((_BLK, d), lambda i: (i, 0)),
            pl.BlockSpec((d, d), lambda i: (0, 0)),
            pl.BlockSpec((_NC * _NS, _BLK), lambda i: (0, i)),
        ],
        out_specs=[
            pl.BlockSpec((_BLK, d), lambda i: (i, 0)),
            pl.BlockSpec((_BLK, 1), lambda i: (i, 0)),
        ],
        out_shape=[
            jax.ShapeDtypeStruct((n, d), jnp.float32),
            jax.ShapeDtypeStruct((n, 1), jnp.float32),
        ],
    )(x, w1, parts)


def _tc_mid(acc, g1, dis, b1, w2):
    _, n, d = acc.shape
    grid = n // _BLK

    def body(a_ref, g_ref, dis_ref, b_ref, w_ref, o_ref):
        comb = a_ref[0] + a_ref[1] + g_ref[...]
        dis = dis_ref[...]
        z = jnp.maximum(comb * dis + b_ref[...], 0.0)
        o_ref[...] = _mm(z, w_ref[...]) * dis

    return pl.pallas_call(
        body,
        grid=(grid,),
        in_specs=[
            pl.BlockSpec((2, _BLK, d), lambda i: (0, i, 0)),
            pl.BlockSpec((_BLK, d), lambda i: (i, 0)),
            pl.BlockSpec((_BLK, 1), lambda i: (i, 0)),
            pl.BlockSpec((1, d), lambda i: (0, 0)),
            pl.BlockSpec((d, d), lambda i: (0, 0)),
        ],
        out_specs=pl.BlockSpec((_BLK, d), lambda i: (i, 0)),
        out_shape=jax.ShapeDtypeStruct((n, d), jnp.float32),
    )(acc, g1, dis, b1, w2)


def _tc_final(acc, g2, dis, b2, fc1w, fc1b, fc2w, fc2b):
    _, n, d = acc.shape
    dout = fc2w.shape[1]
    grid = n // _BLK

    def body(a_ref, g_ref, dis_ref, b_ref, w1_ref, b1_ref, w2_ref, b2_ref,
             zs_ref, res_ref):
        comb = a_ref[0] + a_ref[1] + g_ref[...]
        zs = comb * dis_ref[...] + b_ref[...]
        zs_ref[...] = zs
        hh = _mm(zs, w1_ref[...]) + b1_ref[...]
        h = jnp.where(hh > 0, hh, jnp.exp(jnp.minimum(hh, 0.0)) - 1.0)
        t = _mm(h, w2_ref[...]) + b2_ref[...]
        m = jnp.max(t, axis=1, keepdims=True)
        lse = m + jnp.log(jnp.sum(jnp.exp(t - m), axis=1, keepdims=True))
        res_ref[...] = t - lse

    return pl.pallas_call(
        body,
        grid=(grid,),
        in_specs=[
            pl.BlockSpec((2, _BLK, d), lambda i: (0, i, 0)),
            pl.BlockSpec((_BLK, d), lambda i: (i, 0)),
            pl.BlockSpec((_BLK, 1), lambda i: (i, 0)),
            pl.BlockSpec((1, d), lambda i: (0, 0)),
            pl.BlockSpec((d, d), lambda i: (0, 0)),
            pl.BlockSpec((1, d), lambda i: (0, 0)),
            pl.BlockSpec((d, dout), lambda i: (0, 0)),
            pl.BlockSpec((1, dout), lambda i: (0, 0)),
        ],
        out_specs=[
            pl.BlockSpec((_BLK, d), lambda i: (i, 0)),
            pl.BlockSpec((_BLK, dout), lambda i: (i, 0)),
        ],
        out_shape=[
            jax.ShapeDtypeStruct((n, d), jnp.float32),
            jax.ShapeDtypeStruct((n, dout), jnp.float32),
        ],
    )(acc, g2, dis, b2, fc1w, fc1b, fc2w, fc2b)


def kernel(x, edge_index, W1, b1, W2, b2, fc1_W, fc1_b, fc2_W, fc2_b):
    n, d = x.shape
    e = edge_index.shape[1]
    n_pad = -(-n // (_NS * _CH)) * (_NS * _CH)

    align = _NC * _NS * _CH * _IDXB
    ep = -(-e // align) * align
    pad = ep - e
    row = jnp.concatenate([edge_index[0], jnp.zeros((pad,), edge_index.dtype)])
    col = jnp.concatenate([edge_index[1], jnp.full((pad,), n, edge_index.dtype)])
    row2d = row.reshape(-1, _CH)
    col2d = col.reshape(-1, _CH)
    x_p = jnp.concatenate([x, jnp.zeros((n_pad - n, d), x.dtype)], axis=0)

    parts = _sc_degree(col2d, n_pad)
    g1, dis = _tc_prep(x_p, W1, parts)
    acc1 = _sc_propagate(row2d, col2d, g1, n_pad)
    g2 = _tc_mid(acc1, g1, dis, b1.reshape(1, -1), W2)
    acc2 = _sc_propagate(row2d, col2d, g2, n_pad)
    zs, res = _tc_final(acc2, g2, dis, b2.reshape(1, -1), fc1_W,
                        fc1_b.reshape(1, -1), fc2_W, fc2_b.reshape(1, -1))
    return (zs[:n], res[:n])


# speedup vs baseline: 8.8431x; 1.0509x over previous
import dataclasses
import functools

import jax
import jax.numpy as jnp
from jax import lax
from jax.experimental import pallas as pl
from jax.experimental.pallas import tpu as pltpu
from jax.experimental.pallas import tpu_sc as plsc

_NC = 2
_NS = 16
_LN = 16
_CH = 128
_IDXB = 16
_BLK = 2048
_NBUF = 2


def _sc_compiler_params():
    cp = pltpu.CompilerParams()
    if "needs_layout_passes" in pltpu.CompilerParams.__dataclass_fields__:
        cp = dataclasses.replace(cp, needs_layout_passes=False)
    return cp


def _mm(a, b):
    return lax.dot_general(a, b, (((1,), (0,)), ((), ())),
                           precision=lax.Precision.HIGHEST,
                           preferred_element_type=jnp.float32)


def _sc_degree(col2d, n_pad):
    chunks = col2d.shape[0]
    per_tile = chunks // (_NC * _NS)
    mesh = plsc.VectorSubcoreMesh(core_axis_name="c", subcore_axis_name="s")

    @functools.partial(
        pl.kernel,
        out_type=jax.ShapeDtypeStruct((_NC * _NS, n_pad), jnp.float32),
        mesh=mesh,
        compiler_params=_sc_compiler_params(),
        scratch_types=[
            pltpu.VMEM((n_pad,), jnp.float32),
            pltpu.VMEM((_IDXB, _CH), jnp.int32),
        ],
    )
    def k(col_hbm, out_hbm, hist, idx):
        c = lax.axis_index("c")
        s = lax.axis_index("s")
        wid = s * _NC + c
        zeros = jnp.zeros((_LN,), jnp.float32)
        ones = jnp.ones((_LN,), jnp.float32)

        @pl.loop(0, n_pad // _LN)
        def _(i):
            hist[pl.ds(i * _LN, _LN)] = zeros

        base = wid * per_tile

        @pl.loop(0, per_tile // _IDXB)
        def _(b):
            pltpu.sync_copy(col_hbm.at[pl.ds(base + b * _IDXB, _IDXB)], idx)

            @pl.loop(0, _IDXB)
            def _(j):
                @pl.loop(0, _CH // _LN)
                def _(q):
                    idx16 = idx[j, pl.ds(q * _LN, _LN)]
                    plsc.addupdate_scatter(hist, [idx16], ones)

        pltpu.sync_copy(hist, out_hbm.at[wid])

    return k(col2d)


def _sc_propagate(row2d, col2d, g, n, n_pad):
    chunks = row2d.shape[0]
    per_tile = chunks // (_NC * _NS)
    d = g.shape[1]
    arows = -(-n // (_NS * 8)) * 8
    na = _NS * arows
    afull = arows // _CH
    arem = arows - afull * _CH
    mesh = plsc.VectorSubcoreMesh(core_axis_name="c", subcore_axis_name="s")

    @functools.partial(
        pl.kernel,
        out_type=jax.ShapeDtypeStruct((_NC, n_pad, d), jnp.float32),
        mesh=mesh,
        scratch_types=[
            pltpu.VMEM((_IDXB, _CH), jnp.int32),
            pltpu.VMEM((_IDXB, _CH), jnp.int32),
            pltpu.VMEM((2, _CH, d), jnp.float32),
            pltpu.VMEM_SHARED((na, d), jnp.float32),
            pltpu.SemaphoreType.DMA,
            pltpu.SemaphoreType.DMA,
        ],
    )
    def k(row_hbm, col_hbm, g_hbm, out_hbm, ridx, cidx, gbufs, acc, gsem, ssem):
        c = lax.axis_index("c")
        s = lax.axis_index("s")
        zeros = jnp.zeros((_LN,), jnp.float32)

        base = (s * _NC + c) * per_tile

        zbuf = gbufs.at[0]

        @pl.loop(0, _CH)
        def _(i):
            @pl.loop(0, d // _LN)
            def _(j):
                zbuf[i, pl.ds(j * _LN, _LN)] = zeros

        @pl.loop(0, afull)
        def _(i):
            pltpu.sync_copy(zbuf, acc.at[pl.ds(s * arows + i * _CH, _CH)])

        if arem:
            pltpu.sync_copy(
                zbuf.at[pl.ds(0, arem)],
                acc.at[pl.ds(s * arows + afull * _CH, arem)])

        plsc.subcore_barrier()

        def fire_gather(buf, chunk_ref):
            return pltpu.async_copy(g_hbm.at[chunk_ref], gbufs.at[buf], gsem)

        def fire_scatter(buf, chunk_ref):
            return pltpu.async_copy(gbufs.at[buf], acc.at[chunk_ref], ssem,
                                    add=True)

        pltpu.sync_copy(row_hbm.at[pl.ds(base, _IDXB)], ridx)
        pltpu.sync_copy(col_hbm.at[pl.ds(base, _IDXB)], cidx)
        fire_gather(0, ridx.at[0])

        @pl.loop(0, per_tile // _IDXB)
        def _(bb):
            for j in range(_IDXB):
                if j > 1:
                    pltpu.make_async_copy(
                        gbufs.at[j % 2], acc.at[cidx.at[0]], ssem).wait()
                pltpu.make_async_copy(
                    g_hbm.at[ridx.at[0]], gbufs.at[j % 2], gsem).wait()
                fire_scatter(j % 2, cidx.at[j])
                if j + 1 < _IDXB:
                    fire_gather((j + 1) % 2, ridx.at[j + 1])

            for j in range(2):
                pltpu.make_async_copy(
                    gbufs.at[j], acc.at[cidx.at[0]], ssem).wait()

            @pl.when(bb + 1 < per_tile // _IDXB)
            def _():
                nxt = base + (bb + 1) * _IDXB
                pltpu.sync_copy(row_hbm.at[pl.ds(nxt, _IDXB)], ridx)
                pltpu.sync_copy(col_hbm.at[pl.ds(nxt, _IDXB)], cidx)
                fire_gather(0, ridx.at[0])

        plsc.subcore_barrier()
        pltpu.sync_copy(acc.at[pl.ds(s * arows, arows)],
                        out_hbm.at[c].at[pl.ds(s * arows, arows)])

    return k(row2d, col2d, g)


def _tc_prep(x, w1, parts):
    n, d = x.shape
    grid = n // _BLK

    def body(x_ref, w_ref, p_ref, g_ref, dis_ref):
        deg = jnp.sum(p_ref[...], axis=0) + 1.0
        dis = lax.rsqrt(deg)
        g_ref[...] = _mm(x_ref[...], w_ref[...]) * dis[:, None]
        dis_ref[...] = dis[:, None]

    return pl.pallas_call(
        body,
        grid=(grid,),
        in_specs=[
            pl.BlockSpec((_BLK, d), lambda i: (i, 0)),
            pl.BlockSpec((d, d), lambda i: (0, 0)),
            pl.BlockSpec((_NC * _NS, _BLK), lambda i: (0, i)),
        ],
        out_specs=[
            pl.BlockSpec((_BLK, d), lambda i: (i, 0)),
            pl.BlockSpec((_BLK, 1), lambda i: (i, 0)),
        ],
        out_shape=[
            jax.ShapeDtypeStruct((n, d), jnp.float32),
            jax.ShapeDtypeStruct((n, 1), jnp.float32),
        ],
    )(x, w1, parts)


def _tc_mid(acc, g1, dis, b1, w2):
    _, n, d = acc.shape
    grid = n // _BLK

    def body(a_ref, g_ref, dis_ref, b_ref, w_ref, o_ref):
        comb = a_ref[0] + a_ref[1] + g_ref[...]
        dis = dis_ref[...]
        z = jnp.maximum(comb * dis + b_ref[...], 0.0)
        o_ref[...] = _mm(z, w_ref[...]) * dis

    return pl.pallas_call(
        body,
        grid=(grid,),
        in_specs=[
            pl.BlockSpec((2, _BLK, d), lambda i: (0, i, 0)),
            pl.BlockSpec((_BLK, d), lambda i: (i, 0)),
            pl.BlockSpec((_BLK, 1), lambda i: (i, 0)),
            pl.BlockSpec((1, d), lambda i: (0, 0)),
            pl.BlockSpec((d, d), lambda i: (0, 0)),
        ],
        out_specs=pl.BlockSpec((_BLK, d), lambda i: (i, 0)),
        out_shape=jax.ShapeDtypeStruct((n, d), jnp.float32),
    )(acc, g1, dis, b1, w2)


def _tc_final(acc, g2, dis, b2, fc1w, fc1b, fc2w, fc2b):
    _, n, d = acc.shape
    dout = fc2w.shape[1]
    grid = n // _BLK

    def body(a_ref, g_ref, dis_ref, b_ref, w1_ref, b1_ref, w2_ref, b2_ref,
             zs_ref, res_ref):
        comb = a_ref[0] + a_ref[1] + g_ref[...]
        zs = comb * dis_ref[...] + b_ref[...]
        zs_ref[...] = zs
        hh = _mm(zs, w1_ref[...]) + b1_ref[...]
        h = jnp.where(hh > 0, hh, jnp.exp(jnp.minimum(hh, 0.0)) - 1.0)
        t = _mm(h, w2_ref[...]) + b2_ref[...]
        m = jnp.max(t, axis=1, keepdims=True)
        lse = m + jnp.log(jnp.sum(jnp.exp(t - m), axis=1, keepdims=True))
        res_ref[...] = t - lse

    return pl.pallas_call(
        body,
        grid=(grid,),
        in_specs=[
            pl.BlockSpec((2, _BLK, d), lambda i: (0, i, 0)),
            pl.BlockSpec((_BLK, d), lambda i: (i, 0)),
            pl.BlockSpec((_BLK, 1), lambda i: (i, 0)),
            pl.BlockSpec((1, d), lambda i: (0, 0)),
            pl.BlockSpec((d, d), lambda i: (0, 0)),
            pl.BlockSpec((1, d), lambda i: (0, 0)),
            pl.BlockSpec((d, dout), lambda i: (0, 0)),
            pl.BlockSpec((1, dout), lambda i: (0, 0)),
        ],
        out_specs=[
            pl.BlockSpec((_BLK, d), lambda i: (i, 0)),
            pl.BlockSpec((_BLK, dout), lambda i: (i, 0)),
        ],
        out_shape=[
            jax.ShapeDtypeStruct((n, d), jnp.float32),
            jax.ShapeDtypeStruct((n, dout), jnp.float32),
        ],
    )(acc, g2, dis, b2, fc1w, fc1b, fc2w, fc2b)


def kernel(x, edge_index, W1, b1, W2, b2, fc1_W, fc1_b, fc2_W, fc2_b):
    n, d = x.shape
    e = edge_index.shape[1]
    n_pad = -(-n // (_NS * _CH)) * (_NS * _CH)

    align = _NC * _NS * _CH * _IDXB
    ep = -(-e // align) * align
    pad = ep - e
    row = jnp.concatenate([edge_index[0], jnp.zeros((pad,), edge_index.dtype)])
    col = jnp.concatenate([edge_index[1], jnp.full((pad,), n, edge_index.dtype)])
    row2d = row.reshape(-1, _CH)
    col2d = col.reshape(-1, _CH)
    x_p = jnp.concatenate([x, jnp.zeros((n_pad - n, d), x.dtype)], axis=0)

    parts = _sc_degree(col2d, n_pad)
    g1, dis = _tc_prep(x_p, W1, parts)
    acc1 = _sc_propagate(row2d, col2d, g1, n, n_pad)
    g2 = _tc_mid(acc1, g1, dis, b1.reshape(1, -1), W2)
    acc2 = _sc_propagate(row2d, col2d, g2, n, n_pad)
    zs, res = _tc_final(acc2, g2, dis, b2.reshape(1, -1), fc1_W,
                        fc1_b.reshape(1, -1), fc2_W, fc2_b.reshape(1, -1))
    return (zs[:n], res[:n])


# speedup vs baseline: 8.8502x; 1.0008x over previous
import dataclasses
import functools

import jax
import jax.numpy as jnp
from jax import lax
from jax.experimental import pallas as pl
from jax.experimental.pallas import tpu as pltpu
from jax.experimental.pallas import tpu_sc as plsc

_NC = 2
_NS = 16
_LN = 16
_CH = 128
_IDXB = 16
_BLK = 2048
_NBUF = 2


def _sc_compiler_params():
    cp = pltpu.CompilerParams()
    if "needs_layout_passes" in pltpu.CompilerParams.__dataclass_fields__:
        cp = dataclasses.replace(cp, needs_layout_passes=False)
    return cp


def _mm(a, b):
    return lax.dot_general(a, b, (((1,), (0,)), ((), ())),
                           precision=lax.Precision.HIGHEST,
                           preferred_element_type=jnp.float32)


def _sc_degree(col2d, n_pad):
    chunks = col2d.shape[0]
    per_tile = chunks // (_NC * _NS)
    mesh = plsc.VectorSubcoreMesh(core_axis_name="c", subcore_axis_name="s")

    @functools.partial(
        pl.kernel,
        out_type=jax.ShapeDtypeStruct((_NC * _NS, n_pad), jnp.float32),
        mesh=mesh,
        compiler_params=_sc_compiler_params(),
        scratch_types=[
            pltpu.VMEM((n_pad,), jnp.float32),
            pltpu.VMEM((_IDXB, _CH), jnp.int32),
        ],
    )
    def k(col_hbm, out_hbm, hist, idx):
        c = lax.axis_index("c")
        s = lax.axis_index("s")
        wid = s * _NC + c
        zeros = jnp.zeros((_LN,), jnp.float32)
        ones = jnp.ones((_LN,), jnp.float32)

        @pl.loop(0, n_pad // _LN)
        def _(i):
            hist[pl.ds(i * _LN, _LN)] = zeros

        base = wid * per_tile

        @pl.loop(0, per_tile // _IDXB)
        def _(b):
            pltpu.sync_copy(col_hbm.at[pl.ds(base + b * _IDXB, _IDXB)], idx)

            @pl.loop(0, _IDXB)
            def _(j):
                @pl.loop(0, _CH // _LN)
                def _(q):
                    idx16 = idx[j, pl.ds(q * _LN, _LN)]
                    plsc.addupdate_scatter(hist, [idx16], ones)

        pltpu.sync_copy(hist, out_hbm.at[wid])

    return k(col2d)


def _sc_propagate(row2d, col2d, g, n, n_pad):
    chunks = row2d.shape[0]
    per_tile = chunks // (_NC * _NS)
    d = g.shape[1]
    arows = -(-n // (_NS * 8)) * 8
    na = _NS * arows
    afull = arows // _CH
    arem = arows - afull * _CH
    mesh = plsc.VectorSubcoreMesh(core_axis_name="c", subcore_axis_name="s")

    @functools.partial(
        pl.kernel,
        out_type=jax.ShapeDtypeStruct((_NC, n_pad, d), jnp.float32),
        mesh=mesh,
        scratch_types=[
            pltpu.VMEM((_IDXB, _CH), jnp.int32),
            pltpu.VMEM((_IDXB, _CH), jnp.int32),
            pltpu.VMEM((2, _CH, d), jnp.float32),
            pltpu.VMEM_SHARED((na, d), jnp.float32),
            pltpu.SemaphoreType.DMA,
            pltpu.SemaphoreType.DMA,
        ],
    )
    def k(row_hbm, col_hbm, g_hbm, out_hbm, ridx, cidx, gbufs, acc, gsem, ssem):
        c = lax.axis_index("c")
        s = lax.axis_index("s")
        zeros = jnp.zeros((_LN,), jnp.float32)

        base = (s * _NC + c) * per_tile

        zbuf = gbufs.at[0]

        @pl.loop(0, _CH)
        def _(i):
            @pl.loop(0, d // _LN)
            def _(j):
                zbuf[i, pl.ds(j * _LN, _LN)] = zeros

        @pl.loop(0, afull)
        def _(i):
            pltpu.sync_copy(zbuf, acc.at[pl.ds(s * arows + i * _CH, _CH)])

        if arem:
            pltpu.sync_copy(
                zbuf.at[pl.ds(0, arem)],
                acc.at[pl.ds(s * arows + afull * _CH, arem)])

        plsc.subcore_barrier()

        def fire_gather(buf, chunk_ref):
            return pltpu.async_copy(g_hbm.at[chunk_ref], gbufs.at[buf], gsem)

        def fire_scatter(buf, chunk_ref):
            return pltpu.async_copy(gbufs.at[buf], acc.at[chunk_ref], ssem,
                                    add=True)

        pltpu.sync_copy(row_hbm.at[pl.ds(base, _IDXB)], ridx)
        pltpu.sync_copy(col_hbm.at[pl.ds(base, _IDXB)], cidx)
        fire_gather(0, ridx.at[0])

        @pl.loop(0, per_tile // _IDXB)
        def _(bb):
            for j in range(_IDXB):
                pltpu.make_async_copy(
                    g_hbm.at[ridx.at[0]], gbufs.at[j % 2], gsem).wait()
                fire_scatter(j % 2, cidx.at[j])
                if j + 1 < _IDXB:
                    if j > 0:
                        pltpu.make_async_copy(
                            gbufs.at[(j + 1) % 2], acc.at[cidx.at[0]],
                            ssem).wait()
                    fire_gather((j + 1) % 2, ridx.at[j + 1])

            pltpu.make_async_copy(
                gbufs.at[(_IDXB - 1) % 2], acc.at[cidx.at[0]], ssem).wait()

            @pl.when(bb + 1 < per_tile // _IDXB)
            def _():
                nxt = base + (bb + 1) * _IDXB
                pltpu.sync_copy(row_hbm.at[pl.ds(nxt, _IDXB)], ridx)
                pltpu.sync_copy(col_hbm.at[pl.ds(nxt, _IDXB)], cidx)
                fire_gather(0, ridx.at[0])

        plsc.subcore_barrier()
        pltpu.sync_copy(acc.at[pl.ds(s * arows, arows)],
                        out_hbm.at[c].at[pl.ds(s * arows, arows)])

    return k(row2d, col2d, g)


def _tc_prep(x, w1, parts):
    n, d = x.shape
    grid = n // _BLK

    def body(x_ref, w_ref, p_ref, g_ref, dis_ref):
        deg = jnp.sum(p_ref[...], axis=0) + 1.0
        dis = lax.rsqrt(deg)
        g_ref[...] = _mm(x_ref[...], w_ref[...]) * dis[:, None]
        dis_ref[...] = dis[:, None]

    return pl.pallas_call(
        body,
        grid=(grid,),
        in_specs=[
            pl.BlockSpec((_BLK, d), lambda i: (i, 0)),
            pl.BlockSpec((d, d), lambda i: (0, 0)),
            pl.BlockSpec((_NC * _NS, _BLK), lambda i: (0, i)),
        ],
        out_specs=[
            pl.BlockSpec((_BLK, d), lambda i: (i, 0)),
            pl.BlockSpec((_BLK, 1), lambda i: (i, 0)),
        ],
        out_shape=[
            jax.ShapeDtypeStruct((n, d), jnp.float32),
            jax.ShapeDtypeStruct((n, 1), jnp.float32),
        ],
    )(x, w1, parts)


def _tc_mid(acc, g1, dis, b1, w2):
    _, n, d = acc.shape
    grid = n // _BLK

    def body(a_ref, g_ref, dis_ref, b_ref, w_ref, o_ref):
        comb = a_ref[0] + a_ref[1] + g_ref[...]
        dis = dis_ref[...]
        z = jnp.maximum(comb * dis + b_ref[...], 0.0)
        o_ref[...] = _mm(z, w_ref[...]) * dis

    return pl.pallas_call(
        body,
        grid=(grid,),
        in_specs=[
            pl.BlockSpec((2, _BLK, d), lambda i: (0, i, 0)),
            pl.BlockSpec((_BLK, d), lambda i: (i, 0)),
            pl.BlockSpec((_BLK, 1), lambda i: (i, 0)),
            pl.BlockSpec((1, d), lambda i: (0, 0)),
            pl.BlockSpec((d, d), lambda i: (0, 0)),
        ],
        out_specs=pl.BlockSpec((_BLK, d), lambda i: (i, 0)),
        out_shape=jax.ShapeDtypeStruct((n, d), jnp.float32),
    )(acc, g1, dis, b1, w2)


def _tc_final(acc, g2, dis, b2, fc1w, fc1b, fc2w, fc2b):
    _, n, d = acc.shape
    dout = fc2w.shape[1]
    grid = n // _BLK

    def body(a_ref, g_ref, dis_ref, b_ref, w1_ref, b1_ref, w2_ref, b2_ref,
             zs_ref, res_ref):
        comb = a_ref[0] + a_ref[1] + g_ref[...]
        zs = comb * dis_ref[...] + b_ref[...]
        zs_ref[...] = zs
        hh = _mm(zs, w1_ref[...]) + b1_ref[...]
        h = jnp.where(hh > 0, hh, jnp.exp(jnp.minimum(hh, 0.0)) - 1.0)
        t = _mm(h, w2_ref[...]) + b2_ref[...]
        m = jnp.max(t, axis=1, keepdims=True)
        lse = m + jnp.log(jnp.sum(jnp.exp(t - m), axis=1, keepdims=True))
        res_ref[...] = t - lse

    return pl.pallas_call(
        body,
        grid=(grid,),
        in_specs=[
            pl.BlockSpec((2, _BLK, d), lambda i: (0, i, 0)),
            pl.BlockSpec((_BLK, d), lambda i: (i, 0)),
            pl.BlockSpec((_BLK, 1), lambda i: (i, 0)),
            pl.BlockSpec((1, d), lambda i: (0, 0)),
            pl.BlockSpec((d, d), lambda i: (0, 0)),
            pl.BlockSpec((1, d), lambda i: (0, 0)),
            pl.BlockSpec((d, dout), lambda i: (0, 0)),
            pl.BlockSpec((1, dout), lambda i: (0, 0)),
        ],
        out_specs=[
            pl.BlockSpec((_BLK, d), lambda i: (i, 0)),
            pl.BlockSpec((_BLK, dout), lambda i: (i, 0)),
        ],
        out_shape=[
            jax.ShapeDtypeStruct((n, d), jnp.float32),
            jax.ShapeDtypeStruct((n, dout), jnp.float32),
        ],
    )(acc, g2, dis, b2, fc1w, fc1b, fc2w, fc2b)


def kernel(x, edge_index, W1, b1, W2, b2, fc1_W, fc1_b, fc2_W, fc2_b):
    n, d = x.shape
    e = edge_index.shape[1]
    n_pad = -(-n // (_NS * _CH)) * (_NS * _CH)

    align = _NC * _NS * _CH * _IDXB
    ep = -(-e // align) * align
    pad = ep - e
    row = jnp.concatenate([edge_index[0], jnp.zeros((pad,), edge_index.dtype)])
    col = jnp.concatenate([edge_index[1], jnp.full((pad,), n, edge_index.dtype)])
    row2d = row.reshape(-1, _CH)
    col2d = col.reshape(-1, _CH)
    x_p = jnp.concatenate([x, jnp.zeros((n_pad - n, d), x.dtype)], axis=0)

    parts = _sc_degree(col2d, n_pad)
    g1, dis = _tc_prep(x_p, W1, parts)
    acc1 = _sc_propagate(row2d, col2d, g1, n, n_pad)
    g2 = _tc_mid(acc1, g1, dis, b1.reshape(1, -1), W2)
    acc2 = _sc_propagate(row2d, col2d, g2, n, n_pad)
    zs, res = _tc_final(acc2, g2, dis, b2.reshape(1, -1), fc1_W,
                        fc1_b.reshape(1, -1), fc2_W, fc2_b.reshape(1, -1))
    return (zs[:n], res[:n])


# speedup vs baseline: 10.0045x; 1.1304x over previous
import dataclasses
import functools

import jax
import jax.numpy as jnp
from jax import lax
from jax.experimental import pallas as pl
from jax.experimental.pallas import tpu as pltpu
from jax.experimental.pallas import tpu_sc as plsc

_NC = 2
_NS = 16
_LN = 16
_CH = 128
_IDXB = 16
_BLK = 2048
_NBUF = 2


def _sc_compiler_params():
    cp = pltpu.CompilerParams()
    if "needs_layout_passes" in pltpu.CompilerParams.__dataclass_fields__:
        cp = dataclasses.replace(cp, needs_layout_passes=False)
    return cp


def _mm(a, b):
    return lax.dot_general(a, b, (((1,), (0,)), ((), ())),
                           precision=lax.Precision.HIGHEST,
                           preferred_element_type=jnp.float32)


def _sc_degree(col2d, n_pad):
    chunks = col2d.shape[0]
    per_tile = chunks // (_NC * _NS)
    mesh = plsc.VectorSubcoreMesh(core_axis_name="c", subcore_axis_name="s")

    @functools.partial(
        pl.kernel,
        out_type=jax.ShapeDtypeStruct((_NC * _NS, n_pad), jnp.float32),
        mesh=mesh,
        compiler_params=_sc_compiler_params(),
        scratch_types=[
            pltpu.VMEM((n_pad,), jnp.float32),
            pltpu.VMEM((_IDXB, _CH), jnp.int32),
        ],
    )
    def k(col_hbm, out_hbm, hist, idx):
        c = lax.axis_index("c")
        s = lax.axis_index("s")
        wid = s * _NC + c
        zeros = jnp.zeros((_LN,), jnp.float32)
        ones = jnp.ones((_LN,), jnp.float32)

        @pl.loop(0, n_pad // _LN)
        def _(i):
            hist[pl.ds(i * _LN, _LN)] = zeros

        base = wid * per_tile

        @pl.loop(0, per_tile // _IDXB)
        def _(b):
            pltpu.sync_copy(col_hbm.at[pl.ds(base + b * _IDXB, _IDXB)], idx)

            @pl.loop(0, _IDXB)
            def _(j):
                @pl.loop(0, _CH // _LN)
                def _(q):
                    idx16 = idx[j, pl.ds(q * _LN, _LN)]
                    plsc.addupdate_scatter(hist, [idx16], ones)

        pltpu.sync_copy(hist, out_hbm.at[wid])

    return k(col2d)


def _sc_propagate(row2d, col2d, g, n, n_pad):
    chunks = row2d.shape[0]
    nblocks = chunks // _IDXB
    blocks0 = max(_NS, (nblocks * 4 // 5) // _NS * _NS)
    nb0 = blocks0 // _NS
    nb1 = (nblocks - blocks0) // _NS
    d = g.shape[1]
    arows = -(-n // (_NS * 8)) * 8
    na = _NS * arows
    afull = arows // _CH
    arem = arows - afull * _CH
    mesh = plsc.VectorSubcoreMesh(core_axis_name="c", subcore_axis_name="s")

    @functools.partial(
        pl.kernel,
        out_type=jax.ShapeDtypeStruct((_NC, n_pad, d), jnp.float32),
        mesh=mesh,
        scratch_types=[
            pltpu.VMEM((_IDXB, _CH), jnp.int32),
            pltpu.VMEM((_IDXB, _CH), jnp.int32),
            pltpu.VMEM((2, _CH, d), jnp.float32),
            pltpu.VMEM_SHARED((na, d), jnp.float32),
            pltpu.SemaphoreType.DMA,
            pltpu.SemaphoreType.DMA,
        ],
    )
    def k(row_hbm, col_hbm, g_hbm, out_hbm, ridx, cidx, gbufs, acc, gsem, ssem):
        c = lax.axis_index("c")
        s = lax.axis_index("s")
        zeros = jnp.zeros((_LN,), jnp.float32)

        nblk = jnp.where(c == 0, nb0, nb1)
        base = jnp.where(c == 0, s * nb0, blocks0 + s * nb1) * _IDXB

        zbuf = gbufs.at[0]

        @pl.loop(0, _CH)
        def _(i):
            @pl.loop(0, d // _LN)
            def _(j):
                zbuf[i, pl.ds(j * _LN, _LN)] = zeros

        @pl.loop(0, afull)
        def _(i):
            pltpu.sync_copy(zbuf, acc.at[pl.ds(s * arows + i * _CH, _CH)])

        if arem:
            pltpu.sync_copy(
                zbuf.at[pl.ds(0, arem)],
                acc.at[pl.ds(s * arows + afull * _CH, arem)])

        plsc.subcore_barrier()

        def fire_gather(buf, chunk_ref):
            return pltpu.async_copy(g_hbm.at[chunk_ref], gbufs.at[buf], gsem)

        def fire_scatter(buf, chunk_ref):
            return pltpu.async_copy(gbufs.at[buf], acc.at[chunk_ref], ssem,
                                    add=True)

        pltpu.sync_copy(row_hbm.at[pl.ds(base, _IDXB)], ridx)
        pltpu.sync_copy(col_hbm.at[pl.ds(base, _IDXB)], cidx)
        fire_gather(0, ridx.at[0])

        @pl.loop(0, nblk)
        def _(bb):
            for j in range(_IDXB):
                pltpu.make_async_copy(
                    g_hbm.at[ridx.at[0]], gbufs.at[j % 2], gsem).wait()
                fire_scatter(j % 2, cidx.at[j])
                if j + 1 < _IDXB:
                    if j > 0:
                        pltpu.make_async_copy(
                            gbufs.at[(j + 1) % 2], acc.at[cidx.at[0]],
                            ssem).wait()
                    fire_gather((j + 1) % 2, ridx.at[j + 1])

            pltpu.make_async_copy(
                gbufs.at[(_IDXB - 1) % 2], acc.at[cidx.at[0]], ssem).wait()

            @pl.when(bb + 1 < nblk)
            def _():
                nxt = base + (bb + 1) * _IDXB
                pltpu.sync_copy(row_hbm.at[pl.ds(nxt, _IDXB)], ridx)
                pltpu.sync_copy(col_hbm.at[pl.ds(nxt, _IDXB)], cidx)
                fire_gather(0, ridx.at[0])

        plsc.subcore_barrier()
        pltpu.sync_copy(acc.at[pl.ds(s * arows, arows)],
                        out_hbm.at[c].at[pl.ds(s * arows, arows)])

    return k(row2d, col2d, g)


def _tc_prep(x, w1, parts):
    n, d = x.shape
    grid = n // _BLK

    def body(x_ref, w_ref, p_ref, g_ref, dis_ref):
        deg = jnp.sum(p_ref[...], axis=0) + 1.0
        dis = lax.rsqrt(deg)
        g_ref[...] = _mm(x_ref[...], w_ref[...]) * dis[:, None]
        dis_ref[...] = dis[:, None]

    return pl.pallas_call(
        body,
        grid=(grid,),
        in_specs=[
            pl.BlockSpec((_BLK, d), lambda i: (i, 0)),
            pl.BlockSpec((d, d), lambda i: (0, 0)),
            pl.BlockSpec((_NC * _NS, _BLK), lambda i: (0, i)),
        ],
        out_specs=[
            pl.BlockSpec((_BLK, d), lambda i: (i, 0)),
            pl.BlockSpec((_BLK, 1), lambda i: (i, 0)),
        ],
        out_shape=[
            jax.ShapeDtypeStruct((n, d), jnp.float32),
            jax.ShapeDtypeStruct((n, 1), jnp.float32),
        ],
    )(x, w1, parts)


def _tc_mid(acc, g1, dis, b1, w2):
    _, n, d = acc.shape
    grid = n // _BLK

    def body(a_ref, g_ref, dis_ref, b_ref, w_ref, o_ref):
        comb = a_ref[0] + a_ref[1] + g_ref[...]
        dis = dis_ref[...]
        z = jnp.maximum(comb * dis + b_ref[...], 0.0)
        o_ref[...] = _mm(z, w_ref[...]) * dis

    return pl.pallas_call(
        body,
        grid=(grid,),
        in_specs=[
            pl.BlockSpec((2, _BLK, d), lambda i: (0, i, 0)),
            pl.BlockSpec((_BLK, d), lambda i: (i, 0)),
            pl.BlockSpec((_BLK, 1), lambda i: (i, 0)),
            pl.BlockSpec((1, d), lambda i: (0, 0)),
            pl.BlockSpec((d, d), lambda i: (0, 0)),
        ],
        out_specs=pl.BlockSpec((_BLK, d), lambda i: (i, 0)),
        out_shape=jax.ShapeDtypeStruct((n, d), jnp.float32),
    )(acc, g1, dis, b1, w2)


def _tc_final(acc, g2, dis, b2, fc1w, fc1b, fc2w, fc2b):
    _, n, d = acc.shape
    dout = fc2w.shape[1]
    grid = n // _BLK

    def body(a_ref, g_ref, dis_ref, b_ref, w1_ref, b1_ref, w2_ref, b2_ref,
             zs_ref, res_ref):
        comb = a_ref[0] + a_ref[1] + g_ref[...]
        zs = comb * dis_ref[...] + b_ref[...]
        zs_ref[...] = zs
        hh = _mm(zs, w1_ref[...]) + b1_ref[...]
        h = jnp.where(hh > 0, hh, jnp.exp(jnp.minimum(hh, 0.0)) - 1.0)
        t = _mm(h, w2_ref[...]) + b2_ref[...]
        m = jnp.max(t, axis=1, keepdims=True)
        lse = m + jnp.log(jnp.sum(jnp.exp(t - m), axis=1, keepdims=True))
        res_ref[...] = t - lse

    return pl.pallas_call(
        body,
        grid=(grid,),
        in_specs=[
            pl.BlockSpec((2, _BLK, d), lambda i: (0, i, 0)),
            pl.BlockSpec((_BLK, d), lambda i: (i, 0)),
            pl.BlockSpec((_BLK, 1), lambda i: (i, 0)),
            pl.BlockSpec((1, d), lambda i: (0, 0)),
            pl.BlockSpec((d, d), lambda i: (0, 0)),
            pl.BlockSpec((1, d), lambda i: (0, 0)),
            pl.BlockSpec((d, dout), lambda i: (0, 0)),
            pl.BlockSpec((1, dout), lambda i: (0, 0)),
        ],
        out_specs=[
            pl.BlockSpec((_BLK, d), lambda i: (i, 0)),
            pl.BlockSpec((_BLK, dout), lambda i: (i, 0)),
        ],
        out_shape=[
            jax.ShapeDtypeStruct((n, d), jnp.float32),
            jax.ShapeDtypeStruct((n, dout), jnp.float32),
        ],
    )(acc, g2, dis, b2, fc1w, fc1b, fc2w, fc2b)


def kernel(x, edge_index, W1, b1, W2, b2, fc1_W, fc1_b, fc2_W, fc2_b):
    n, d = x.shape
    e = edge_index.shape[1]
    n_pad = -(-n // (_NS * _CH)) * (_NS * _CH)

    align = _NC * _NS * _CH * _IDXB
    ep = -(-e // align) * align
    pad = ep - e
    row = jnp.concatenate([edge_index[0], jnp.zeros((pad,), edge_index.dtype)])
    col = jnp.concatenate([edge_index[1], jnp.full((pad,), n, edge_index.dtype)])
    row2d = row.reshape(-1, _CH)
    col2d = col.reshape(-1, _CH)
    x_p = jnp.concatenate([x, jnp.zeros((n_pad - n, d), x.dtype)], axis=0)

    parts = _sc_degree(col2d, n_pad)
    g1, dis = _tc_prep(x_p, W1, parts)
    acc1 = _sc_propagate(row2d, col2d, g1, n, n_pad)
    g2 = _tc_mid(acc1, g1, dis, b1.reshape(1, -1), W2)
    acc2 = _sc_propagate(row2d, col2d, g2, n, n_pad)
    zs, res = _tc_final(acc2, g2, dis, b2.reshape(1, -1), fc1_W,
                        fc1_b.reshape(1, -1), fc2_W, fc2_b.reshape(1, -1))
    return (zs[:n], res[:n])


# speedup vs baseline: 27.2483x; 2.7236x over previous
import dataclasses
import functools

import jax
import jax.numpy as jnp
from jax import lax
from jax.experimental import pallas as pl
from jax.experimental.pallas import tpu as pltpu
from jax.experimental.pallas import tpu_sc as plsc

_NC = 2
_NS = 16
_LN = 16
_CH = 128
_IDXB = 16
_BLK = 2048
_NBUF = 2


def _sc_compiler_params():
    cp = pltpu.CompilerParams()
    if "needs_layout_passes" in pltpu.CompilerParams.__dataclass_fields__:
        cp = dataclasses.replace(cp, needs_layout_passes=False)
    return cp


def _mm(a, b):
    return lax.dot_general(a, b, (((1,), (0,)), ((), ())),
                           precision=lax.Precision.HIGHEST,
                           preferred_element_type=jnp.float32)


def _sc_degree(col2d, n_pad):
    chunks = col2d.shape[0]
    per_tile = chunks // (_NC * _NS)
    mesh = plsc.VectorSubcoreMesh(core_axis_name="c", subcore_axis_name="s")

    @functools.partial(
        pl.kernel,
        out_type=jax.ShapeDtypeStruct((_NC * _NS, n_pad), jnp.float32),
        mesh=mesh,
        compiler_params=_sc_compiler_params(),
        scratch_types=[
            pltpu.VMEM((n_pad,), jnp.float32),
            pltpu.VMEM((_IDXB, _CH), jnp.int32),
        ],
    )
    def k(col_hbm, out_hbm, hist, idx):
        c = lax.axis_index("c")
        s = lax.axis_index("s")
        wid = s * _NC + c
        zeros = jnp.zeros((_LN,), jnp.float32)
        ones = jnp.ones((_LN,), jnp.float32)

        @pl.loop(0, n_pad // _LN)
        def _(i):
            hist[pl.ds(i * _LN, _LN)] = zeros

        base = wid * per_tile

        @pl.loop(0, per_tile // _IDXB)
        def _(b):
            pltpu.sync_copy(col_hbm.at[pl.ds(base + b * _IDXB, _IDXB)], idx)

            @pl.loop(0, _IDXB)
            def _(j):
                @pl.loop(0, _CH // _LN)
                def _(q):
                    idx16 = idx[j, pl.ds(q * _LN, _LN)]
                    plsc.addupdate_scatter(hist, [idx16], ones)

        pltpu.sync_copy(hist, out_hbm.at[wid])

    return k(col2d)


def _sc_propagate(row2d, col2d, g, n, n_pad):
    chunks = row2d.shape[0]
    nblocks = chunks // _IDXB
    blocks0 = nblocks // 2
    nb0 = blocks0 // _NS
    nb1 = (nblocks - blocks0) // _NS
    d = g.shape[1]
    arows = -(-n // (_NS * 8)) * 8
    na = _NS * arows
    afull = arows // _CH
    arem = arows - afull * _CH
    mesh = plsc.VectorSubcoreMesh(core_axis_name="c", subcore_axis_name="s")

    @functools.partial(
        pl.kernel,
        out_type=jax.ShapeDtypeStruct((_NC, n_pad, d), jnp.float32),
        mesh=mesh,
        scratch_types=[
            pltpu.VMEM((_IDXB, _CH), jnp.int32),
            pltpu.VMEM((_IDXB, _CH), jnp.int32),
            pltpu.VMEM((2, _CH, d), jnp.float32),
            pltpu.VMEM_SHARED((na, d), jnp.float32),
            pltpu.SemaphoreType.DMA,
            pltpu.SemaphoreType.DMA,
        ],
    )
    def k(row_hbm, col_hbm, g_hbm, out_hbm, ridx, cidx, gbufs, acc, gsem, ssem):
        c = lax.axis_index("c")
        s = lax.axis_index("s")
        zeros = jnp.zeros((_LN,), jnp.float32)

        nblk = jnp.where(c == 0, nb0, nb1)
        base = jnp.where(c == 0, s * nb0, blocks0 + s * nb1) * _IDXB

        zbuf = gbufs.at[0]

        @pl.loop(0, _CH)
        def _(i):
            @pl.loop(0, d // _LN)
            def _(j):
                zbuf[i, pl.ds(j * _LN, _LN)] = zeros

        @pl.loop(0, afull)
        def _(i):
            pltpu.sync_copy(zbuf, acc.at[pl.ds(s * arows + i * _CH, _CH)])

        if arem:
            pltpu.sync_copy(
                zbuf.at[pl.ds(0, arem)],
                acc.at[pl.ds(s * arows + afull * _CH, arem)])

        plsc.subcore_barrier()

        def fire_gather(buf, chunk_ref):
            return pltpu.async_copy(g_hbm.at[chunk_ref], gbufs.at[buf], gsem)

        def fire_scatter(buf, chunk_ref):
            return pltpu.async_copy(gbufs.at[buf], acc.at[chunk_ref], ssem,
                                    add=True)

        pltpu.sync_copy(row_hbm.at[pl.ds(base, _IDXB)], ridx)
        pltpu.sync_copy(col_hbm.at[pl.ds(base, _IDXB)], cidx)
        fire_gather(0, ridx.at[0])

        @pl.loop(0, nblk)
        def _(bb):
            for j in range(_IDXB):
                pltpu.make_async_copy(
                    g_hbm.at[ridx.at[0]], gbufs.at[j % 2], gsem).wait()
                fire_scatter(j % 2, cidx.at[j])
                if j + 1 < _IDXB:
                    if j > 0:
                        pltpu.make_async_copy(
                            gbufs.at[(j + 1) % 2], acc.at[cidx.at[0]],
                            ssem).wait()
                    fire_gather((j + 1) % 2, ridx.at[j + 1])

            pltpu.make_async_copy(
                gbufs.at[(_IDXB - 1) % 2], acc.at[cidx.at[0]], ssem).wait()

            @pl.when(bb + 1 < nblk)
            def _():
                nxt = base + (bb + 1) * _IDXB
                pltpu.sync_copy(row_hbm.at[pl.ds(nxt, _IDXB)], ridx)
                pltpu.sync_copy(col_hbm.at[pl.ds(nxt, _IDXB)], cidx)
                fire_gather(0, ridx.at[0])

        plsc.subcore_barrier()
        pltpu.sync_copy(acc.at[pl.ds(s * arows, arows)],
                        out_hbm.at[c].at[pl.ds(s * arows, arows)])

    return k(row2d, col2d, g)


def _tc_prep(x, w1, parts):
    n, d = x.shape
    grid = n // _BLK

    def body(x_ref, w_ref, p_ref, g_ref, dis_ref):
        deg = jnp.sum(p_ref[...], axis=0) + 1.0
        dis = lax.rsqrt(deg)
        g_ref[...] = _mm(x_ref[...], w_ref[...]) * dis[:, None]
        dis_ref[...] = dis[:, None]

    return pl.pallas_call(
        body,
        grid=(grid,),
        in_specs=[
            pl.BlockSpec((_BLK, d), lambda i: (i, 0)),
            pl.BlockSpec((d, d), lambda i: (0, 0)),
            pl.BlockSpec((_NC * _NS, _BLK), lambda i: (0, i)),
        ],
        out_specs=[
            pl.BlockSpec((_BLK, d), lambda i: (i, 0)),
            pl.BlockSpec((_BLK, 1), lambda i: (i, 0)),
        ],
        out_shape=[
            jax.ShapeDtypeStruct((n, d), jnp.float32),
            jax.ShapeDtypeStruct((n, 1), jnp.float32),
        ],
    )(x, w1, parts)


def _tc_mid(acc, g1, dis, b1, w2):
    _, n, d = acc.shape
    grid = n // _BLK

    def body(a_ref, g_ref, dis_ref, b_ref, w_ref, o_ref):
        comb = a_ref[0] + a_ref[1] + g_ref[...]
        dis = dis_ref[...]
        z = jnp.maximum(comb * dis + b_ref[...], 0.0)
        o_ref[...] = _mm(z, w_ref[...]) * dis

    return pl.pallas_call(
        body,
        grid=(grid,),
        in_specs=[
            pl.BlockSpec((2, _BLK, d), lambda i: (0, i, 0)),
            pl.BlockSpec((_BLK, d), lambda i: (i, 0)),
            pl.BlockSpec((_BLK, 1), lambda i: (i, 0)),
            pl.BlockSpec((1, d), lambda i: (0, 0)),
            pl.BlockSpec((d, d), lambda i: (0, 0)),
        ],
        out_specs=pl.BlockSpec((_BLK, d), lambda i: (i, 0)),
        out_shape=jax.ShapeDtypeStruct((n, d), jnp.float32),
    )(acc, g1, dis, b1, w2)


def _tc_final(acc, g2, dis, b2, fc1w, fc1b, fc2w, fc2b):
    _, n, d = acc.shape
    dout = fc2w.shape[1]
    grid = n // _BLK

    def body(a_ref, g_ref, dis_ref, b_ref, w1_ref, b1_ref, w2_ref, b2_ref,
             zs_ref, res_ref):
        comb = a_ref[0] + a_ref[1] + g_ref[...]
        zs = comb * dis_ref[...] + b_ref[...]
        zs_ref[...] = zs
        hh = _mm(zs, w1_ref[...]) + b1_ref[...]
        h = jnp.where(hh > 0, hh, jnp.exp(jnp.minimum(hh, 0.0)) - 1.0)
        t = _mm(h, w2_ref[...]) + b2_ref[...]
        m = jnp.max(t, axis=1, keepdims=True)
        lse = m + jnp.log(jnp.sum(jnp.exp(t - m), axis=1, keepdims=True))
        res_ref[...] = t - lse

    return pl.pallas_call(
        body,
        grid=(grid,),
        in_specs=[
            pl.BlockSpec((2, _BLK, d), lambda i: (0, i, 0)),
            pl.BlockSpec((_BLK, d), lambda i: (i, 0)),
            pl.BlockSpec((_BLK, 1), lambda i: (i, 0)),
            pl.BlockSpec((1, d), lambda i: (0, 0)),
            pl.BlockSpec((d, d), lambda i: (0, 0)),
            pl.BlockSpec((1, d), lambda i: (0, 0)),
            pl.BlockSpec((d, dout), lambda i: (0, 0)),
            pl.BlockSpec((1, dout), lambda i: (0, 0)),
        ],
        out_specs=[
            pl.BlockSpec((_BLK, d), lambda i: (i, 0)),
            pl.BlockSpec((_BLK, dout), lambda i: (i, 0)),
        ],
        out_shape=[
            jax.ShapeDtypeStruct((n, d), jnp.float32),
            jax.ShapeDtypeStruct((n, dout), jnp.float32),
        ],
    )(acc, g2, dis, b2, fc1w, fc1b, fc2w, fc2b)


def kernel(x, edge_index, W1, b1, W2, b2, fc1_W, fc1_b, fc2_W, fc2_b):
    n, d = x.shape
    e = edge_index.shape[1]
    n_pad = -(-n // (_NS * _CH)) * (_NS * _CH)

    align = _NC * _NS * _CH * _IDXB
    ep = -(-e // align) * align
    pad = ep - e
    arows = -(-n // (_NS * 8)) * 8
    na = _NS * arows
    pad_i = jnp.arange(pad, dtype=edge_index.dtype)
    row = jnp.concatenate([edge_index[0], pad_i % n])
    col = jnp.concatenate([edge_index[1], n + pad_i % (na - n)])
    row2d = row.reshape(-1, _CH)
    col2d = col.reshape(-1, _CH)
    x_p = jnp.concatenate([x, jnp.zeros((n_pad - n, d), x.dtype)], axis=0)

    parts = _sc_degree(col2d, n_pad)
    g1, dis = _tc_prep(x_p, W1, parts)
    acc1 = _sc_propagate(row2d, col2d, g1, n, n_pad)
    g2 = _tc_mid(acc1, g1, dis, b1.reshape(1, -1), W2)
    acc2 = _sc_propagate(row2d, col2d, g2, n, n_pad)
    zs, res = _tc_final(acc2, g2, dis, b2.reshape(1, -1), fc1_W,
                        fc1_b.reshape(1, -1), fc2_W, fc2_b.reshape(1, -1))
    return (zs[:n], res[:n])


# speedup vs baseline: 33.8536x; 1.2424x over previous
import dataclasses
import functools

import jax
import jax.numpy as jnp
from jax import lax
from jax.experimental import pallas as pl
from jax.experimental.pallas import tpu as pltpu
from jax.experimental.pallas import tpu_sc as plsc

_NC = 2
_NS = 16
_LN = 16
_CH = 128
_IDXB = 16
_BLK = 2048
_NBUF = 3
_CHP = 120


def _sc_compiler_params():
    cp = pltpu.CompilerParams()
    if "needs_layout_passes" in pltpu.CompilerParams.__dataclass_fields__:
        cp = dataclasses.replace(cp, needs_layout_passes=False)
    return cp


def _mm(a, b):
    return lax.dot_general(a, b, (((1,), (0,)), ((), ())),
                           precision=lax.Precision.HIGHEST,
                           preferred_element_type=jnp.float32)


def _sc_degree(col2d, n_pad):
    chunks = col2d.shape[0]
    per_tile = chunks // (_NC * _NS)
    mesh = plsc.VectorSubcoreMesh(core_axis_name="c", subcore_axis_name="s")

    @functools.partial(
        pl.kernel,
        out_type=jax.ShapeDtypeStruct((_NC * _NS, n_pad), jnp.float32),
        mesh=mesh,
        compiler_params=_sc_compiler_params(),
        scratch_types=[
            pltpu.VMEM((n_pad,), jnp.float32),
            pltpu.VMEM((_IDXB, _CH), jnp.int32),
        ],
    )
    def k(col_hbm, out_hbm, hist, idx):
        c = lax.axis_index("c")
        s = lax.axis_index("s")
        wid = s * _NC + c
        zeros = jnp.zeros((_LN,), jnp.float32)
        ones = jnp.ones((_LN,), jnp.float32)

        @pl.loop(0, n_pad // _LN)
        def _(i):
            hist[pl.ds(i * _LN, _LN)] = zeros

        base = wid * per_tile

        @pl.loop(0, per_tile // _IDXB)
        def _(b):
            pltpu.sync_copy(col_hbm.at[pl.ds(base + b * _IDXB, _IDXB)], idx)

            @pl.loop(0, _IDXB)
            def _(j):
                @pl.loop(0, _CH // _LN)
                def _(q):
                    idx16 = idx[j, pl.ds(q * _LN, _LN)]
                    plsc.addupdate_scatter(hist, [idx16], ones)

        pltpu.sync_copy(hist, out_hbm.at[wid])

    return k(col2d)


def _sc_propagate(row3d, col3d, g, n, n_pad):
    half_blocks = row3d.shape[0]
    per_tile_hb = half_blocks // (_NC * _NS)
    nT = per_tile_hb // 2
    d = g.shape[1]
    arows = -(-n // (_NS * 8)) * 8
    na = _NS * arows
    afull = arows // _CHP
    arem = arows - afull * _CHP
    mesh = plsc.VectorSubcoreMesh(core_axis_name="c", subcore_axis_name="s")

    @functools.partial(
        pl.kernel,
        out_type=jax.ShapeDtypeStruct((_NC, n_pad, d), jnp.float32),
        mesh=mesh,
        scratch_types=[
            pltpu.VMEM((3, _CHP), jnp.int32),
            pltpu.VMEM((3, _CHP), jnp.int32),
            pltpu.VMEM((3, _CHP), jnp.int32),
            pltpu.VMEM((3, _CHP), jnp.int32),
            pltpu.VMEM((3, _CHP, d), jnp.float32),
            pltpu.VMEM_SHARED((na, d), jnp.float32),
            pltpu.SemaphoreType.DMA,
            pltpu.SemaphoreType.DMA,
        ],
    )
    def k(row_hbm, col_hbm, g_hbm, out_hbm, ridxA, ridxB, cidxA, cidxB,
          gbufs, acc, gsem, ssem):
        c = lax.axis_index("c")
        s = lax.axis_index("s")
        zeros = jnp.zeros((_LN,), jnp.float32)

        zbuf = gbufs.at[0]

        @pl.loop(0, _CHP)
        def _(i):
            @pl.loop(0, d // _LN)
            def _(j):
                zbuf[i, pl.ds(j * _LN, _LN)] = zeros

        @pl.loop(0, afull)
        def _(i):
            pltpu.sync_copy(zbuf,
                            acc.at[pl.ds(s * arows + i * _CHP, _CHP)])

        if arem:
            pltpu.sync_copy(
                zbuf.at[pl.ds(0, arem)],
                acc.at[pl.ds(s * arows + afull * _CHP, arem)])

        plsc.subcore_barrier()

        hb = (s * _NC + c) * per_tile_hb

        def g_fire(b, idxrow):
            pltpu.async_copy(g_hbm.at[idxrow], gbufs.at[b], gsem)

        def g_wait(b):
            pltpu.make_async_copy(g_hbm.at[ridxA.at[0]], gbufs.at[b],
                                  gsem).wait()

        def s_fire(b, idxrow):
            pltpu.async_copy(gbufs.at[b], acc.at[idxrow], ssem, add=True)

        def s_drain(b):
            pltpu.make_async_copy(gbufs.at[b], acc.at[cidxA.at[0]],
                                  ssem).wait()

        pltpu.sync_copy(row_hbm.at[hb], ridxA)
        pltpu.sync_copy(col_hbm.at[hb], cidxA)
        g_fire(0, ridxA.at[0])
        g_fire(1, ridxA.at[1])

        @pl.loop(0, nT)
        def _(t):
            g_wait(0)
            s_fire(0, cidxA.at[0])

            @pl.when(t > 0)
            def _():
                s_drain(2)
            g_fire(2, ridxA.at[2])

            pltpu.sync_copy(row_hbm.at[hb + 2 * t + 1], ridxB)
            pltpu.sync_copy(col_hbm.at[hb + 2 * t + 1], cidxB)

            g_wait(1)
            s_fire(1, cidxA.at[1])
            s_drain(0)
            g_fire(0, ridxB.at[0])

            g_wait(2)
            s_fire(2, cidxA.at[2])
            s_drain(1)
            g_fire(1, ridxB.at[1])

            g_wait(0)
            s_fire(0, cidxB.at[0])
            s_drain(2)
            g_fire(2, ridxB.at[2])

            @pl.when(t + 1 < nT)
            def _():
                pltpu.sync_copy(row_hbm.at[hb + 2 * t + 2], ridxA)
                pltpu.sync_copy(col_hbm.at[hb + 2 * t + 2], cidxA)

            g_wait(1)
            s_fire(1, cidxB.at[1])
            s_drain(0)

            @pl.when(t + 1 < nT)
            def _():
                g_fire(0, ridxA.at[0])

            g_wait(2)
            s_fire(2, cidxB.at[2])
            s_drain(1)

            @pl.when(t + 1 < nT)
            def _():
                g_fire(1, ridxA.at[1])

        s_drain(2)

        plsc.subcore_barrier()
        pltpu.sync_copy(acc.at[pl.ds(s * arows, arows)],
                        out_hbm.at[c].at[pl.ds(s * arows, arows)])

    return k(row3d, col3d, g)


def _tc_prep(x, w1, parts):
    n, d = x.shape
    grid = n // _BLK

    def body(x_ref, w_ref, p_ref, g_ref, dis_ref):
        deg = jnp.sum(p_ref[...], axis=0) + 1.0
        dis = lax.rsqrt(deg)
        g_ref[...] = _mm(x_ref[...], w_ref[...]) * dis[:, None]
        dis_ref[...] = dis[:, None]

    return pl.pallas_call(
        body,
        grid=(grid,),
        in_specs=[
            pl.BlockSpec((_BLK, d), lambda i: (i, 0)),
            pl.BlockSpec((d, d), lambda i: (0, 0)),
            pl.BlockSpec((_NC * _NS, _BLK), lambda i: (0, i)),
        ],
        out_specs=[
            pl.BlockSpec((_BLK, d), lambda i: (i, 0)),
            pl.BlockSpec((_BLK, 1), lambda i: (i, 0)),
        ],
        out_shape=[
            jax.ShapeDtypeStruct((n, d), jnp.float32),
            jax.ShapeDtypeStruct((n, 1), jnp.float32),
        ],
    )(x, w1, parts)


def _tc_mid(acc, g1, dis, b1, w2):
    _, n, d = acc.shape
    grid = n // _BLK

    def body(a_ref, g_ref, dis_ref, b_ref, w_ref, o_ref):
        comb = a_ref[0] + a_ref[1] + g_ref[...]
        dis = dis_ref[...]
        z = jnp.maximum(comb * dis + b_ref[...], 0.0)
        o_ref[...] = _mm(z, w_ref[...]) * dis

    return pl.pallas_call(
        body,
        grid=(grid,),
        in_specs=[
            pl.BlockSpec((2, _BLK, d), lambda i: (0, i, 0)),
            pl.BlockSpec((_BLK, d), lambda i: (i, 0)),
            pl.BlockSpec((_BLK, 1), lambda i: (i, 0)),
            pl.BlockSpec((1, d), lambda i: (0, 0)),
            pl.BlockSpec((d, d), lambda i: (0, 0)),
        ],
        out_specs=pl.BlockSpec((_BLK, d), lambda i: (i, 0)),
        out_shape=jax.ShapeDtypeStruct((n, d), jnp.float32),
    )(acc, g1, dis, b1, w2)


def _tc_final(acc, g2, dis, b2, fc1w, fc1b, fc2w, fc2b):
    _, n, d = acc.shape
    dout = fc2w.shape[1]
    grid = n // _BLK

    def body(a_ref, g_ref, dis_ref, b_ref, w1_ref, b1_ref, w2_ref, b2_ref,
             zs_ref, res_ref):
        comb = a_ref[0] + a_ref[1] + g_ref[...]
        zs = comb * dis_ref[...] + b_ref[...]
        zs_ref[...] = zs
        hh = _mm(zs, w1_ref[...]) + b1_ref[...]
        h = jnp.where(hh > 0, hh, jnp.exp(jnp.minimum(hh, 0.0)) - 1.0)
        t = _mm(h, w2_ref[...]) + b2_ref[...]
        m = jnp.max(t, axis=1, keepdims=True)
        lse = m + jnp.log(jnp.sum(jnp.exp(t - m), axis=1, keepdims=True))
        res_ref[...] = t - lse

    return pl.pallas_call(
        body,
        grid=(grid,),
        in_specs=[
            pl.BlockSpec((2, _BLK, d), lambda i: (0, i, 0)),
            pl.BlockSpec((_BLK, d), lambda i: (i, 0)),
            pl.BlockSpec((_BLK, 1), lambda i: (i, 0)),
            pl.BlockSpec((1, d), lambda i: (0, 0)),
            pl.BlockSpec((d, d), lambda i: (0, 0)),
            pl.BlockSpec((1, d), lambda i: (0, 0)),
            pl.BlockSpec((d, dout), lambda i: (0, 0)),
            pl.BlockSpec((1, dout), lambda i: (0, 0)),
        ],
        out_specs=[
            pl.BlockSpec((_BLK, d), lambda i: (i, 0)),
            pl.BlockSpec((_BLK, dout), lambda i: (i, 0)),
        ],
        out_shape=[
            jax.ShapeDtypeStruct((n, d), jnp.float32),
            jax.ShapeDtypeStruct((n, dout), jnp.float32),
        ],
    )(acc, g2, dis, b2, fc1w, fc1b, fc2w, fc2b)


def kernel(x, edge_index, W1, b1, W2, b2, fc1_W, fc1_b, fc2_W, fc2_b):
    n, d = x.shape
    e = edge_index.shape[1]
    n_pad = -(-n // (_NS * _CH)) * (_NS * _CH)

    align = _NC * _NS * _CH * _IDXB
    ep = -(-e // align) * align
    pad = ep - e
    arows = -(-n // (_NS * 8)) * 8
    na = _NS * arows
    pad_i = jnp.arange(pad, dtype=edge_index.dtype)
    col_deg = jnp.concatenate([edge_index[1], n + pad_i % (n_pad - n)])
    col2d = col_deg.reshape(-1, _CH)

    alignp = _NC * _NS * _CHP * 6
    epp = -(-e // alignp) * alignp
    padp = epp - e
    pad_j = jnp.arange(padp, dtype=edge_index.dtype)
    rowp = jnp.concatenate([edge_index[0], pad_j % n])
    colp = jnp.concatenate([edge_index[1], n + pad_j % (na - n)])
    row3d = rowp.reshape(-1, 3, _CHP)
    col3d = colp.reshape(-1, 3, _CHP)
    x_p = jnp.concatenate([x, jnp.zeros((n_pad - n, d), x.dtype)], axis=0)

    parts = _sc_degree(col2d, n_pad)
    g1, dis = _tc_prep(x_p, W1, parts)
    acc1 = _sc_propagate(row3d, col3d, g1, n, n_pad)
    g2 = _tc_mid(acc1, g1, dis, b1.reshape(1, -1), W2)
    acc2 = _sc_propagate(row3d, col3d, g2, n, n_pad)
    zs, res = _tc_final(acc2, g2, dis, b2.reshape(1, -1), fc1_W,
                        fc1_b.reshape(1, -1), fc2_W, fc2_b.reshape(1, -1))
    return (zs[:n], res[:n])
